# baseline (speedup 1.0000x reference)
import jax
import jax.numpy as jnp
from jax import lax
from jax.experimental import pallas as pl
from jax.experimental.pallas import tpu as pltpu
from jax.experimental.pallas import tpu_sc as plsc

_S = 50
_B = 4096
_D = 300
_SP = 64
_NC = 2
_NS = 16
_NW = _NC * _NS
_COLS = _B // _NW
_PD = 1152
_VP = 25 * 4096


def _sc_pool_body(t0_hbm, t1_hbm, t2_hbm, xt_hbm, xg_hbm, out_hbm,
                  xt_blk, xg_blk,
                  r00, r01, r02, r10, r11, r12,
                  r20, r21, r22, r30, r31, r32, out_blk,
                  sem0, sem1, sem2, sem3):
    wid = lax.axis_index("s") * _NC + lax.axis_index("c")
    base = wid * _COLS
    zeros16 = jnp.zeros((16,), jnp.float32)
    ones16 = jnp.full((16,), 1.0, jnp.float32)

    pltpu.sync_copy(xt_hbm.at[pl.ds(base, _COLS)], xt_blk)
    pltpu.sync_copy(xg_hbm.at[pl.ds(base, _COLS)], xg_blk)
    zoffs = [s + d for s in (0, 384, 768)
             for d in (300, 316, 332, 348, 364, 368)]
    for r in range(16):
        for z in zoffs:
            out_blk[r, pl.ds(z, 16)] = zeros16

    def fetch(c, rows3, sem):
        idx = xg_blk.at[c]
        pltpu.make_async_copy(t0_hbm.at[idx], rows3[0], sem).start()
        pltpu.make_async_copy(t1_hbm.at[idx], rows3[1], sem).start()
        pltpu.make_async_copy(t2_hbm.at[idx], rows3[2], sem).start()

    def compute(c, rows3, sem):
        idx = xg_blk.at[c]
        pltpu.make_async_copy(t0_hbm.at[idx], rows3[0], sem).wait()
        pltpu.make_async_copy(t1_hbm.at[idx], rows3[1], sem).wait()
        pltpu.make_async_copy(t2_hbm.at[idx], rows3[2], sem).wait()
        r = lax.rem(c, 16)
        cnt = jnp.zeros((16,), jnp.float32)
        for k in range(4):
            cnt = cnt + jnp.where(xt_blk[c, pl.ds(16 * k, 16)] != 1,
                                  ones16, zeros16)
        inv = ones16 / jnp.full((16,), jnp.sum(cnt))
        for j in range(19):
            off = 284 if j == 18 else 16 * j
            rows = rows3[off // 128]
            loff = off % 128

            def body(i, carry, rows=rows, loff=loff):
                acc = list(carry)
                for u in range(10):
                    v = rows[i * 10 + u, pl.ds(loff, 16)]
                    k = u % 4
                    acc[k] = acc[k] + v
                    acc[4 + k] = jnp.maximum(acc[4 + k], v)
                    acc[8 + k] = jnp.minimum(acc[8 + k], v)
                return tuple(acc)

            ninf16 = jnp.full((16,), -jnp.inf, jnp.float32)
            pinf16 = jnp.full((16,), jnp.inf, jnp.float32)
            init = (zeros16,) * 4 + (ninf16,) * 4 + (pinf16,) * 4
            acc = lax.fori_loop(0, _S // 10, body, init)
            a_s = (acc[0] + acc[1]) + (acc[2] + acc[3])
            a_mx = jnp.maximum(jnp.maximum(acc[4], acc[5]),
                               jnp.maximum(acc[6], acc[7]))
            a_mn = jnp.minimum(jnp.minimum(acc[8], acc[9]),
                               jnp.minimum(acc[10], acc[11]))
            out_blk[r, pl.ds(off, 16)] = a_mx
            out_blk[r, pl.ds(384 + off, 16)] = a_s * inv
            out_blk[r, pl.ds(768 + off, 16)] = a_mn

    sets = ((r00, r01, r02), (r10, r11, r12),
            (r20, r21, r22), (r30, r31, r32))
    sems = (sem0, sem1, sem2, sem3)
    fetch(0, sets[0], sems[0])
    fetch(1, sets[1], sems[1])

    def loop_body(it, carry):
        for bnum in range(4):
            c = 4 * it + bnum
            nxt = (bnum + 2) % 4

            @pl.when(c + 2 < _COLS)
            def _(c=c, nxt=nxt):
                fetch(c + 2, sets[nxt], sems[nxt])

            compute(c, sets[bnum], sems[bnum])

        @pl.when(lax.rem(it, 4) == 3)
        def _():
            grp = lax.div(it, 4)
            pltpu.sync_copy(out_blk, out_hbm.at[pl.ds(base + grp * 16, 16)])

        return carry

    lax.fori_loop(0, _COLS // 4, loop_body, 0)


def _sc_pool(t0, t1, t2, xtp, xgp):
    mesh = plsc.VectorSubcoreMesh(core_axis_name="c", subcore_axis_name="s")
    f = pl.kernel(
        _sc_pool_body,
        out_type=jax.ShapeDtypeStruct((_B, _PD), jnp.float32),
        mesh=mesh,
        compiler_params=pltpu.CompilerParams(use_tc_tiling_on_sc=False,
                                             needs_layout_passes=False),
        scratch_types=[
            pltpu.VMEM((_COLS, _SP), jnp.int32),
            pltpu.VMEM((_COLS, _S), jnp.int32),
        ] + [pltpu.VMEM((_S, 128), jnp.float32)] * 12 + [
            pltpu.VMEM((16, _PD), jnp.float32),
            pltpu.SemaphoreType.DMA,
            pltpu.SemaphoreType.DMA,
            pltpu.SemaphoreType.DMA,
            pltpu.SemaphoreType.DMA,
        ],
    )
    return f(t0, t1, t2, xtp, xgp)


def _tc_transpose_body(i_ref, o0_ref, o1_ref, o2_ref):
    o0_ref[...] = i_ref[pl.ds(0, 128)].T
    o1_ref[...] = i_ref[pl.ds(128, 128)].T
    o2_ref[...] = jnp.concatenate(
        [i_ref[pl.ds(256, 44)],
         jnp.zeros((84, 4096), jnp.float32)], axis=0).T


def _tc_transpose(tt):
    spec = pl.BlockSpec((4096, 128), lambda i: (i, 0))
    return pl.pallas_call(
        _tc_transpose_body,
        grid=(25,),
        in_specs=[pl.BlockSpec((_D, 4096), lambda i: (0, i))],
        out_specs=[spec, spec, spec],
        out_shape=[jax.ShapeDtypeStruct((_VP, 128), jnp.float32)] * 3,
    )(tt)


def _tc_head_body(p_ref, ag_ref, w1_ref, w2_ref, b_ref, o_ref):
    acc = jnp.dot(p_ref[...], w1_ref[...], preferred_element_type=jnp.float32)
    acc = acc + jnp.dot(ag_ref[...], w2_ref[...],
                        preferred_element_type=jnp.float32)
    acc = acc + b_ref[...]
    cols = lax.broadcasted_iota(jnp.int32, acc.shape, 1)
    acc = jnp.where(cols < 10, acc, -jnp.inf)
    m = jnp.max(acc, axis=1, keepdims=True)
    lse = jnp.log(jnp.sum(jnp.exp(acc - m), axis=1, keepdims=True)) + m
    o_ref[...] = acc - lse


def _tc_head(pooled, ag, w1, w2, bp):
    return pl.pallas_call(
        _tc_head_body,
        grid=(16,),
        in_specs=[
            pl.BlockSpec((_B // 16, _PD), lambda i: (i, 0)),
            pl.BlockSpec((_B // 16, 128), lambda i: (i, 0)),
            pl.BlockSpec((_PD, 128), lambda i: (0, 0)),
            pl.BlockSpec((128, 128), lambda i: (0, 0)),
            pl.BlockSpec((1, 128), lambda i: (0, 0)),
        ],
        out_specs=pl.BlockSpec((_B // 16, 128), lambda i: (i, 0)),
        out_shape=jax.ShapeDtypeStruct((_B, 128), jnp.float32),
    )(pooled, ag, w1, w2, bp)


def kernel(x, age, gender, table, W, b):
    xt = x.T
    xtp = jnp.full((_B, _SP), 1, jnp.int32).at[:, :_S].set(xt)
    t0, t1, t2 = _tc_transpose(jnp.swapaxes(table, 0, 1))
    pooled = _sc_pool(t0, t1, t2, xtp, xt)
    ag = (jnp.zeros((_B, 128), jnp.float32)
          .at[:, :11].set(age).at[:, 11:13].set(gender))
    w1 = (jnp.zeros((_PD, 128), jnp.float32)
          .at[0:300, :10].set(W[:, 0:300].T)
          .at[384:684, :10].set(W[:, 300:600].T)
          .at[768:1068, :10].set(W[:, 600:900].T))
    w2 = jnp.zeros((128, 128), jnp.float32).at[:13, :10].set(W[:, 900:].T)
    bp = jnp.zeros((1, 128), jnp.float32).at[0, :10].set(b)
    out = _tc_head(pooled, ag, w1, w2, bp)
    return out[:, :10]


# speedup vs baseline: 1.0081x; 1.0081x over previous
import jax
import jax.numpy as jnp
from jax import lax
from jax.experimental import pallas as pl
from jax.experimental.pallas import tpu as pltpu
from jax.experimental.pallas import tpu_sc as plsc

_S = 50
_B = 4096
_D = 300
_SP = 64
_NC = 2
_NS = 16
_NW = _NC * _NS
_COLS = _B // _NW
_PD = 1152
_VP = 13 * 8192


def _sc_pool_body(t0_hbm, t1_hbm, t2_hbm, xt_hbm, xg_hbm, out_hbm,
                  xt_blk, xg_blk,
                  r00, r01, r02, r10, r11, r12,
                  r20, r21, r22, r30, r31, r32, out_blk,
                  sem0, sem1, sem2, sem3):
    wid = lax.axis_index("s") * _NC + lax.axis_index("c")
    base = wid * _COLS
    zeros16 = jnp.zeros((16,), jnp.float32)
    ones16 = jnp.full((16,), 1.0, jnp.float32)

    pltpu.sync_copy(xt_hbm.at[pl.ds(base, _COLS)], xt_blk)
    pltpu.sync_copy(xg_hbm.at[pl.ds(base, _COLS)], xg_blk)
    zoffs = [s + d for s in (0, 384, 768)
             for d in (300, 316, 332, 348, 364, 368)]
    for r in range(16):
        for z in zoffs:
            out_blk[r, pl.ds(z, 16)] = zeros16

    def fetch(c, rows3, sem):
        idx = xg_blk.at[c]
        pltpu.make_async_copy(t0_hbm.at[idx], rows3[0], sem).start()
        pltpu.make_async_copy(t1_hbm.at[idx], rows3[1], sem).start()
        pltpu.make_async_copy(t2_hbm.at[idx], rows3[2], sem).start()

    def compute(c, rows3, sem):
        idx = xg_blk.at[c]
        pltpu.make_async_copy(t0_hbm.at[idx], rows3[0], sem).wait()
        pltpu.make_async_copy(t1_hbm.at[idx], rows3[1], sem).wait()
        pltpu.make_async_copy(t2_hbm.at[idx], rows3[2], sem).wait()
        r = lax.rem(c, 16)
        cnt = jnp.zeros((16,), jnp.float32)
        for k in range(4):
            cnt = cnt + jnp.where(xt_blk[c, pl.ds(16 * k, 16)] != 1,
                                  ones16, zeros16)
        inv = ones16 / jnp.full((16,), jnp.sum(cnt))
        for j in range(19):
            off = 284 if j == 18 else 16 * j
            rows = rows3[off // 128]
            loff = off % 128

            def body(i, carry, rows=rows, loff=loff):
                acc = list(carry)
                for u in range(10):
                    v = rows[i * 10 + u, pl.ds(loff, 16)]
                    k = u % 4
                    acc[k] = acc[k] + v
                    acc[4 + k] = jnp.maximum(acc[4 + k], v)
                    acc[8 + k] = jnp.minimum(acc[8 + k], v)
                return tuple(acc)

            ninf16 = jnp.full((16,), -jnp.inf, jnp.float32)
            pinf16 = jnp.full((16,), jnp.inf, jnp.float32)
            init = (zeros16,) * 4 + (ninf16,) * 4 + (pinf16,) * 4
            acc = lax.fori_loop(0, _S // 10, body, init)
            a_s = (acc[0] + acc[1]) + (acc[2] + acc[3])
            a_mx = jnp.maximum(jnp.maximum(acc[4], acc[5]),
                               jnp.maximum(acc[6], acc[7]))
            a_mn = jnp.minimum(jnp.minimum(acc[8], acc[9]),
                               jnp.minimum(acc[10], acc[11]))
            out_blk[r, pl.ds(off, 16)] = a_mx
            out_blk[r, pl.ds(384 + off, 16)] = a_s * inv
            out_blk[r, pl.ds(768 + off, 16)] = a_mn

    sets = ((r00, r01, r02), (r10, r11, r12),
            (r20, r21, r22), (r30, r31, r32))
    sems = (sem0, sem1, sem2, sem3)
    fetch(0, sets[0], sems[0])
    fetch(1, sets[1], sems[1])

    def loop_body(it, carry):
        for bnum in range(4):
            c = 4 * it + bnum
            nxt = (bnum + 2) % 4

            @pl.when(c + 2 < _COLS)
            def _(c=c, nxt=nxt):
                fetch(c + 2, sets[nxt], sems[nxt])

            compute(c, sets[bnum], sems[bnum])

        @pl.when(lax.rem(it, 4) == 3)
        def _():
            grp = lax.div(it, 4)
            pltpu.sync_copy(out_blk, out_hbm.at[pl.ds(base + grp * 16, 16)])

        return carry

    lax.fori_loop(0, _COLS // 4, loop_body, 0)


def _sc_pool(t0, t1, t2, xtp, xgp):
    mesh = plsc.VectorSubcoreMesh(core_axis_name="c", subcore_axis_name="s")
    f = pl.kernel(
        _sc_pool_body,
        out_type=jax.ShapeDtypeStruct((_B, _PD), jnp.float32),
        mesh=mesh,
        compiler_params=pltpu.CompilerParams(use_tc_tiling_on_sc=False,
                                             needs_layout_passes=False),
        scratch_types=[
            pltpu.VMEM((_COLS, _SP), jnp.int32),
            pltpu.VMEM((_COLS, _S), jnp.int32),
        ] + [pltpu.VMEM((_S, 128), jnp.float32)] * 12 + [
            pltpu.VMEM((16, _PD), jnp.float32),
            pltpu.SemaphoreType.DMA,
            pltpu.SemaphoreType.DMA,
            pltpu.SemaphoreType.DMA,
            pltpu.SemaphoreType.DMA,
        ],
    )
    return f(t0, t1, t2, xtp, xgp)


def _tc_transpose_body(i_ref, o0_ref, o1_ref, o2_ref):
    o0_ref[...] = i_ref[pl.ds(0, 128)].T
    o1_ref[...] = i_ref[pl.ds(128, 128)].T
    o2_ref[...] = jnp.concatenate(
        [i_ref[pl.ds(256, 44)],
         jnp.zeros((84, 8192), jnp.float32)], axis=0).T


def _tc_transpose(tt):
    spec = pl.BlockSpec((8192, 128), lambda i: (i, 0))
    return pl.pallas_call(
        _tc_transpose_body,
        grid=(13,),
        in_specs=[pl.BlockSpec((_D, 8192), lambda i: (0, i))],
        out_specs=[spec, spec, spec],
        out_shape=[jax.ShapeDtypeStruct((_VP, 128), jnp.float32)] * 3,
    )(tt)


def _tc_head_body(p_ref, ag_ref, w1_ref, w2_ref, b_ref, o_ref):
    acc = jnp.dot(p_ref[...], w1_ref[...], preferred_element_type=jnp.float32)
    acc = acc + jnp.dot(ag_ref[...], w2_ref[...],
                        preferred_element_type=jnp.float32)
    acc = acc + b_ref[...]
    cols = lax.broadcasted_iota(jnp.int32, acc.shape, 1)
    acc = jnp.where(cols < 10, acc, -jnp.inf)
    m = jnp.max(acc, axis=1, keepdims=True)
    lse = jnp.log(jnp.sum(jnp.exp(acc - m), axis=1, keepdims=True)) + m
    o_ref[...] = acc - lse


def _tc_head(pooled, ag, w1, w2, bp):
    return pl.pallas_call(
        _tc_head_body,
        grid=(16,),
        in_specs=[
            pl.BlockSpec((_B // 16, _PD), lambda i: (i, 0)),
            pl.BlockSpec((_B // 16, 128), lambda i: (i, 0)),
            pl.BlockSpec((_PD, 128), lambda i: (0, 0)),
            pl.BlockSpec((128, 128), lambda i: (0, 0)),
            pl.BlockSpec((1, 128), lambda i: (0, 0)),
        ],
        out_specs=pl.BlockSpec((_B // 16, 128), lambda i: (i, 0)),
        out_shape=jax.ShapeDtypeStruct((_B, 128), jnp.float32),
    )(pooled, ag, w1, w2, bp)


def kernel(x, age, gender, table, W, b):
    xt = x.T
    xtp = jnp.full((_B, _SP), 1, jnp.int32).at[:, :_S].set(xt)
    t0, t1, t2 = _tc_transpose(jnp.swapaxes(table, 0, 1))
    pooled = _sc_pool(t0, t1, t2, xtp, xt)
    ag = (jnp.zeros((_B, 128), jnp.float32)
          .at[:, :11].set(age).at[:, 11:13].set(gender))
    w1 = (jnp.zeros((_PD, 128), jnp.float32)
          .at[0:300, :10].set(W[:, 0:300].T)
          .at[384:684, :10].set(W[:, 300:600].T)
          .at[768:1068, :10].set(W[:, 600:900].T))
    w2 = jnp.zeros((128, 128), jnp.float32).at[:13, :10].set(W[:, 900:].T)
    bp = jnp.zeros((1, 128), jnp.float32).at[0, :10].set(b)
    out = _tc_head(pooled, ag, w1, w2, bp)
    return out[:, :10]
